# Initial kernel scaffold; baseline (speedup 1.0000x reference)
import jax
import jax.numpy as jnp
from jax.experimental import pallas as pl


def kernel(src_x, dst_x, src_batch, dst_batch, bn_weight, bn_bias):
    raise NotImplementedError("write your pallas kernel here")


# speedup vs baseline: 4.3470x; 4.3470x over previous
import functools

import jax
import jax.numpy as jnp
from jax import lax
from jax.experimental import pallas as pl
from jax.experimental.pallas import tpu as pltpu
from jax.experimental.pallas import tpu_sc as plsc

_K = 5
_EPS = 1e-5
_BQ = 128
_NC, _NS = 2, 16
_NW = _NC * _NS


def _topk_body(src_ref, dst_ref, idx_ref):
    bq = src_ref.shape[0]
    n = dst_ref.shape[0]
    s = src_ref[...]
    d = dst_ref[...]
    mm = lax.dot_general(s, d, (((1,), (1,)), ((), ())),
                         preferred_element_type=jnp.float32)
    s2 = jnp.sum(s * s, axis=1, keepdims=True)
    d2 = jnp.sum(d * d, axis=1)
    cur = -(s2 - 2.0 * mm + d2[None, :])
    iota = lax.broadcasted_iota(jnp.int32, (bq, n), 1)
    cols = []
    for k in range(_K):
        m = jnp.max(cur, axis=1, keepdims=True)
        cand = jnp.where(cur == m, iota, jnp.int32(n))
        idx = jnp.min(cand, axis=1, keepdims=True)
        cols.append(idx)
        if k < _K - 1:
            cur = jnp.where(iota == idx, -jnp.inf, cur)
    idx_ref[...] = jnp.concatenate(cols, axis=1)


def _tc_topk(src_x, dst_x):
    q, dim = src_x.shape
    n = dst_x.shape[0]
    return pl.pallas_call(
        _topk_body,
        grid=(q // _BQ,),
        in_specs=[
            pl.BlockSpec((_BQ, dim), lambda i: (i, 0)),
            pl.BlockSpec((n, dim), lambda i: (0, 0)),
        ],
        out_specs=pl.BlockSpec((_BQ, _K), lambda i: (i, 0)),
        out_shape=jax.ShapeDtypeStruct((q, _K), jnp.int32),
    )(src_x, dst_x)


def _rsqrt16(x):
    i = plsc.bitcast(x, jnp.int32)
    i = jnp.int32(0x5F3759DF) - lax.shift_right_logical(i, 1)
    y = plsc.bitcast(i, jnp.float32)
    for _ in range(4):
        y = y * (1.5 - 0.5 * x * y * y)
    return y


def _recip16(x):
    i = jnp.int32(0x7EF311C3) - plsc.bitcast(x, jnp.int32)
    y = plsc.bitcast(i, jnp.float32)
    for _ in range(4):
        y = y * (2.0 - x * y)
    return y


@functools.lru_cache(maxsize=None)
def _sc_kernels(n_edges, n_src, n_dst, dim):
    epw = n_edges // _NW
    qpw = n_src // _NW
    rows_per_w = epw // 128
    mesh = plsc.VectorSubcoreMesh(core_axis_name="c", subcore_axis_name="s",
                                  num_cores=_NC)

    @functools.partial(
        pl.kernel, mesh=mesh,
        compiler_params=pltpu.CompilerParams(needs_layout_passes=False),
        out_type=jax.ShapeDtypeStruct((n_edges,), jnp.float32),
        scratch_types=[
            pltpu.VMEM((rows_per_w, 128), jnp.int32),
            pltpu.VMEM((epw, 128), jnp.float32),
            pltpu.VMEM((qpw, dim), jnp.float32),
            pltpu.VMEM((epw,), jnp.float32),
            pltpu.SemaphoreType.DMA,
        ])
    def gather_dot(src_hbm, dst_hbm, idx_hbm, out_hbm,
                   idx_v, rows_v, src_v, dots_v, sem):
        w = lax.axis_index("s") * _NC + lax.axis_index("c")
        pltpu.sync_copy(idx_hbm.at[w], idx_v)
        copies = [
            pltpu.async_copy(dst_hbm.at[idx_v.at[j]],
                             rows_v.at[pl.ds(j * 128, 128)], sem)
            for j in range(rows_per_w)
        ]
        pltpu.sync_copy(src_hbm.at[pl.ds(w * qpw, qpw)], src_v)
        for c in copies:
            c.wait()
        iota16 = lax.iota(jnp.int32, 16)

        def group(g, carry):
            e = g * 16 + iota16
            qloc = e // _K
            acc = jnp.zeros((16,), jnp.float32)
            for f in range(dim):
                fv = jnp.full((16,), f, jnp.int32)
                a = plsc.load_gather(rows_v, [e, fv])
                b = plsc.load_gather(src_v, [qloc, fv])
                acc = acc + a * b
            dots_v[pl.ds(g * 16, 16)] = acc
            return carry

        lax.fori_loop(0, epw // 16, group, 0)
        pltpu.sync_copy(dots_v, out_hbm.at[pl.ds(w * epw, epw)])

    @functools.partial(
        pl.kernel, mesh=mesh,
        compiler_params=pltpu.CompilerParams(needs_layout_passes=False),
        out_type=jax.ShapeDtypeStruct((n_edges,), jnp.float32),
        scratch_types=[
            pltpu.VMEM((n_edges,), jnp.float32),
            pltpu.VMEM((n_edges,), jnp.float32),
            pltpu.VMEM((16,), jnp.float32),
            pltpu.VMEM((16,), jnp.float32),
        ])
    def bn_exp_norm(dots_hbm, w_hbm, b_hbm, out_hbm, x_v, y_v, w_v, b_v):
        wid = lax.axis_index("s") * _NC + lax.axis_index("c")
        groups = n_edges // 16
        inv_n = jnp.float32(1.0 / n_edges)

        @pl.when(wid == 0)
        def _():
            pltpu.sync_copy(dots_hbm, x_v)
            pltpu.sync_copy(w_hbm, w_v)
            pltpu.sync_copy(b_hbm, b_v)

            def p_sum(i, s):
                return s + x_v[pl.ds(i * 16, 16)]

            total = jnp.sum(lax.fori_loop(
                0, groups, p_sum, jnp.zeros((16,), jnp.float32)))
            mean = jnp.full((16,), total, jnp.float32) * inv_n

            def p_var(i, s):
                v = x_v[pl.ds(i * 16, 16)] - mean
                return s + v * v

            vtot = jnp.sum(lax.fori_loop(
                0, groups, p_var, jnp.zeros((16,), jnp.float32)))
            var = jnp.full((16,), vtot, jnp.float32) * inv_n
            scale = _rsqrt16(var + _EPS) * w_v[...]
            bias = b_v[...]

            def p_exp(i, s):
                yv = jnp.exp((x_v[pl.ds(i * 16, 16)] - mean) * scale + bias)
                y_v[pl.ds(i * 16, 16)] = yv
                return s + yv

            es = lax.fori_loop(0, groups, p_exp, jnp.zeros((16,), jnp.float32))
            emean = jnp.full((16,), jnp.sum(es), jnp.float32) * inv_n
            winv = _recip16(emean)

            def p_norm(i, carry):
                y_v[pl.ds(i * 16, 16)] = y_v[pl.ds(i * 16, 16)] * winv
                return carry

            lax.fori_loop(0, groups, p_norm, 0)
            pltpu.sync_copy(y_v, out_hbm)

    return gather_dot, bn_exp_norm


def kernel(src_x, dst_x, src_batch, dst_batch, bn_weight, bn_bias):
    q, dim = src_x.shape
    n = dst_x.shape[0]
    nn_idx = _tc_topk(src_x, dst_x)
    dst_idx = nn_idx.reshape(-1)
    src_idx = jnp.repeat(jnp.arange(q, dtype=jnp.int32), _K)
    edge_index = jnp.stack([src_idx, dst_idx], axis=0)

    gather_dot, bn_exp_norm = _sc_kernels(q * _K, q, n, dim)
    idx2d = dst_idx.reshape(_NW, -1, 128)
    dst_pad = jnp.pad(dst_x, ((0, 0), (0, 128 - dim)))
    dots = gather_dot(src_x, dst_pad, idx2d)
    w16 = jnp.tile(bn_weight, 16)
    b16 = jnp.tile(bn_bias, 16)
    ew = bn_exp_norm(dots, w16, b16)
    return edge_index, ew


# speedup vs baseline: 4.8355x; 1.1124x over previous
import functools

import jax
import jax.numpy as jnp
from jax import lax
from jax.experimental import pallas as pl
from jax.experimental.pallas import tpu as pltpu
from jax.experimental.pallas import tpu_sc as plsc

_K = 5
_EPS = 1e-5
_BQ = 128
_NC, _NS = 2, 16
_NW = _NC * _NS


_W = 128


def _topk_body(src_ref, dst_ref, idx_ref):
    bq = src_ref.shape[0]
    n = dst_ref.shape[0]
    nc = n // _W
    s = src_ref[...]
    d = dst_ref[...]
    mm = lax.dot_general(s, d, (((1,), (1,)), ((), ())),
                         preferred_element_type=jnp.float32)
    s2 = jnp.sum(s * s, axis=1, keepdims=True)
    d2 = jnp.sum(d * d, axis=1)
    cur = -(s2 - 2.0 * mm + d2[None, :])

    ciota = lax.broadcasted_iota(jnp.int32, (bq, nc), 1)
    wiota = lax.broadcasted_iota(jnp.int32, (bq, _W), 1)
    neg_inf = jnp.float32(-jnp.inf)
    m_mat = jnp.full((bq, nc), neg_inf)
    for c in range(nc):
        cmax = jnp.max(cur[:, c * _W:(c + 1) * _W], axis=1, keepdims=True)
        m_mat = jnp.where(ciota == c, cmax, m_mat)

    cols = []
    prev = []
    for k in range(_K):
        m = jnp.max(m_mat, axis=1, keepdims=True)
        cstar = jnp.min(jnp.where(m_mat == m, ciota, jnp.int32(nc)),
                        axis=1, keepdims=True)
        cv = jnp.zeros((bq, _W), jnp.float32)
        for c in range(nc):
            cv = cv + jnp.where(cstar == c, cur[:, c * _W:(c + 1) * _W], 0.0)
        for (cj, pj) in prev:
            cv = jnp.where((cj == cstar) & (wiota == pj), neg_inf, cv)
        pos = jnp.min(jnp.where(cv == m, wiota, jnp.int32(_W)),
                      axis=1, keepdims=True)
        cols.append(cstar * _W + pos)
        prev.append((cstar, pos))
        if k < _K - 1:
            cv = jnp.where(wiota == pos, neg_inf, cv)
            newmax = jnp.max(cv, axis=1, keepdims=True)
            m_mat = jnp.where(ciota == cstar, newmax, m_mat)
    idx_ref[...] = jnp.concatenate(cols, axis=1)


def _tc_topk(src_x, dst_x):
    q, dim = src_x.shape
    n = dst_x.shape[0]
    return pl.pallas_call(
        _topk_body,
        grid=(q // _BQ,),
        in_specs=[
            pl.BlockSpec((_BQ, dim), lambda i: (i, 0)),
            pl.BlockSpec((n, dim), lambda i: (0, 0)),
        ],
        out_specs=pl.BlockSpec((_BQ, _K), lambda i: (i, 0)),
        out_shape=jax.ShapeDtypeStruct((q, _K), jnp.int32),
    )(src_x, dst_x)


def _rsqrt16(x):
    i = plsc.bitcast(x, jnp.int32)
    i = jnp.int32(0x5F3759DF) - lax.shift_right_logical(i, 1)
    y = plsc.bitcast(i, jnp.float32)
    for _ in range(4):
        y = y * (1.5 - 0.5 * x * y * y)
    return y


def _recip16(x):
    i = jnp.int32(0x7EF311C3) - plsc.bitcast(x, jnp.int32)
    y = plsc.bitcast(i, jnp.float32)
    for _ in range(4):
        y = y * (2.0 - x * y)
    return y


@functools.lru_cache(maxsize=None)
def _sc_kernels(n_edges, n_src, n_dst, dim):
    epw = n_edges // _NW
    qpw = n_src // _NW
    rows_per_w = epw // 128
    mesh = plsc.VectorSubcoreMesh(core_axis_name="c", subcore_axis_name="s",
                                  num_cores=_NC)

    @functools.partial(
        pl.kernel, mesh=mesh,
        compiler_params=pltpu.CompilerParams(needs_layout_passes=False),
        out_type=jax.ShapeDtypeStruct((n_edges,), jnp.float32),
        scratch_types=[
            pltpu.VMEM((rows_per_w, 128), jnp.int32),
            pltpu.VMEM((epw, 128), jnp.float32),
            pltpu.VMEM((qpw, dim), jnp.float32),
            pltpu.VMEM((epw,), jnp.float32),
            pltpu.SemaphoreType.DMA,
        ])
    def gather_dot(src_hbm, dst_hbm, idx_hbm, out_hbm,
                   idx_v, rows_v, src_v, dots_v, sem):
        w = lax.axis_index("s") * _NC + lax.axis_index("c")
        pltpu.sync_copy(idx_hbm.at[w], idx_v)
        copies = [
            pltpu.async_copy(dst_hbm.at[idx_v.at[j]],
                             rows_v.at[pl.ds(j * 128, 128)], sem)
            for j in range(rows_per_w)
        ]
        pltpu.sync_copy(src_hbm.at[pl.ds(w * qpw, qpw)], src_v)
        for c in copies:
            c.wait()
        iota16 = lax.iota(jnp.int32, 16)

        def group(g, carry):
            e = g * 16 + iota16
            qloc = e // _K
            acc = jnp.zeros((16,), jnp.float32)
            for f in range(dim):
                fv = jnp.full((16,), f, jnp.int32)
                a = plsc.load_gather(rows_v, [e, fv])
                b = plsc.load_gather(src_v, [qloc, fv])
                acc = acc + a * b
            dots_v[pl.ds(g * 16, 16)] = acc
            return carry

        lax.fori_loop(0, epw // 16, group, 0)
        pltpu.sync_copy(dots_v, out_hbm.at[pl.ds(w * epw, epw)])

    @functools.partial(
        pl.kernel, mesh=mesh,
        compiler_params=pltpu.CompilerParams(needs_layout_passes=False),
        out_type=jax.ShapeDtypeStruct((n_edges,), jnp.float32),
        scratch_types=[
            pltpu.VMEM((n_edges,), jnp.float32),
            pltpu.VMEM((n_edges,), jnp.float32),
            pltpu.VMEM((16,), jnp.float32),
            pltpu.VMEM((16,), jnp.float32),
        ])
    def bn_exp_norm(dots_hbm, w_hbm, b_hbm, out_hbm, x_v, y_v, w_v, b_v):
        wid = lax.axis_index("s") * _NC + lax.axis_index("c")
        groups = n_edges // 16
        inv_n = jnp.float32(1.0 / n_edges)

        @pl.when(wid == 0)
        def _():
            pltpu.sync_copy(dots_hbm, x_v)
            pltpu.sync_copy(w_hbm, w_v)
            pltpu.sync_copy(b_hbm, b_v)

            def p_sum(i, s):
                return s + x_v[pl.ds(i * 16, 16)]

            total = jnp.sum(lax.fori_loop(
                0, groups, p_sum, jnp.zeros((16,), jnp.float32)))
            mean = jnp.full((16,), total, jnp.float32) * inv_n

            def p_var(i, s):
                v = x_v[pl.ds(i * 16, 16)] - mean
                return s + v * v

            vtot = jnp.sum(lax.fori_loop(
                0, groups, p_var, jnp.zeros((16,), jnp.float32)))
            var = jnp.full((16,), vtot, jnp.float32) * inv_n
            scale = _rsqrt16(var + _EPS) * w_v[...]
            bias = b_v[...]

            def p_exp(i, s):
                yv = jnp.exp((x_v[pl.ds(i * 16, 16)] - mean) * scale + bias)
                y_v[pl.ds(i * 16, 16)] = yv
                return s + yv

            es = lax.fori_loop(0, groups, p_exp, jnp.zeros((16,), jnp.float32))
            emean = jnp.full((16,), jnp.sum(es), jnp.float32) * inv_n
            winv = _recip16(emean)

            def p_norm(i, carry):
                y_v[pl.ds(i * 16, 16)] = y_v[pl.ds(i * 16, 16)] * winv
                return carry

            lax.fori_loop(0, groups, p_norm, 0)
            pltpu.sync_copy(y_v, out_hbm)

    return gather_dot, bn_exp_norm


def kernel(src_x, dst_x, src_batch, dst_batch, bn_weight, bn_bias):
    q, dim = src_x.shape
    n = dst_x.shape[0]
    nn_idx = _tc_topk(src_x, dst_x)
    dst_idx = nn_idx.reshape(-1)
    src_idx = jnp.repeat(jnp.arange(q, dtype=jnp.int32), _K)
    edge_index = jnp.stack([src_idx, dst_idx], axis=0)

    gather_dot, bn_exp_norm = _sc_kernels(q * _K, q, n, dim)
    idx2d = dst_idx.reshape(_NW, -1, 128)
    dst_pad = jnp.pad(dst_x, ((0, 0), (0, 128 - dim)))
    dots = gather_dot(src_x, dst_pad, idx2d)
    w16 = jnp.tile(bn_weight, 16)
    b16 = jnp.tile(bn_bias, 16)
    ew = bn_exp_norm(dots, w16, b16)
    return edge_index, ew


# speedup vs baseline: 5.5153x; 1.1406x over previous
import functools

import jax
import jax.numpy as jnp
from jax import lax
from jax.experimental import pallas as pl
from jax.experimental.pallas import tpu as pltpu
from jax.experimental.pallas import tpu_sc as plsc

_K = 5
_EPS = 1e-5
_BQ = 256
_NC, _NS = 2, 16
_NW = _NC * _NS


_W = 128


def _topk_body(src_ref, dst_ref, idx_ref):
    bq = src_ref.shape[0]
    n = dst_ref.shape[0]
    nc = n // _W
    s = src_ref[...]
    d = dst_ref[...]
    mm = lax.dot_general(s, d, (((1,), (1,)), ((), ())),
                         preferred_element_type=jnp.float32)
    s2 = jnp.sum(s * s, axis=1, keepdims=True)
    d2 = jnp.sum(d * d, axis=1)
    cur = s2 - 2.0 * mm + d2[None, :]

    ciota = lax.broadcasted_iota(jnp.int32, (bq, nc), 1)
    wiota = lax.broadcasted_iota(jnp.int32, (bq, _W), 1)
    pos_inf = jnp.float32(jnp.inf)
    m_mat = jnp.full((bq, nc), pos_inf)
    for c in range(nc):
        cmin = jnp.min(cur[:, c * _W:(c + 1) * _W], axis=1, keepdims=True)
        m_mat = jnp.where(ciota == c, cmin, m_mat)

    cols = []
    prev = []
    for k in range(_K):
        m = jnp.min(m_mat, axis=1, keepdims=True)
        cstar = jnp.min(jnp.where(m_mat == m, ciota, jnp.int32(nc)),
                        axis=1, keepdims=True)
        cv = jnp.zeros((bq, _W), jnp.float32)
        for c in range(nc):
            cv = cv + jnp.where(cstar == c, cur[:, c * _W:(c + 1) * _W], 0.0)
        for (cj, pj) in prev:
            cv = jnp.where((cj == cstar) & (wiota == pj), pos_inf, cv)
        pos = jnp.min(jnp.where(cv == m, wiota, jnp.int32(_W)),
                      axis=1, keepdims=True)
        cols.append(cstar * _W + pos)
        prev.append((cstar, pos))
        if k < _K - 1:
            cv = jnp.where(wiota == pos, pos_inf, cv)
            newmin = jnp.min(cv, axis=1, keepdims=True)
            m_mat = jnp.where(ciota == cstar, newmin, m_mat)
    idx_ref[...] = jnp.concatenate(cols, axis=1)


def _tc_topk(src_x, dst_x):
    q, dim = src_x.shape
    n = dst_x.shape[0]
    return pl.pallas_call(
        _topk_body,
        grid=(q // _BQ,),
        in_specs=[
            pl.BlockSpec((_BQ, dim), lambda i: (i, 0)),
            pl.BlockSpec((n, dim), lambda i: (0, 0)),
        ],
        out_specs=pl.BlockSpec((_BQ, _K), lambda i: (i, 0)),
        out_shape=jax.ShapeDtypeStruct((q, _K), jnp.int32),
    )(src_x, dst_x)


def _rsqrt16(x):
    i = plsc.bitcast(x, jnp.int32)
    i = jnp.int32(0x5F3759DF) - lax.shift_right_logical(i, 1)
    y = plsc.bitcast(i, jnp.float32)
    for _ in range(4):
        y = y * (1.5 - 0.5 * x * y * y)
    return y


def _recip16(x):
    i = jnp.int32(0x7EF311C3) - plsc.bitcast(x, jnp.int32)
    y = plsc.bitcast(i, jnp.float32)
    for _ in range(4):
        y = y * (2.0 - x * y)
    return y


@functools.lru_cache(maxsize=None)
def _sc_kernels(n_edges, n_src, n_dst, dim):
    epw = n_edges // _NW
    qpw = n_src // _NW
    rows_per_w = epw // 128
    mesh = plsc.VectorSubcoreMesh(core_axis_name="c", subcore_axis_name="s",
                                  num_cores=_NC)

    @functools.partial(
        pl.kernel, mesh=mesh,
        compiler_params=pltpu.CompilerParams(needs_layout_passes=False),
        out_type=jax.ShapeDtypeStruct((n_edges,), jnp.float32),
        scratch_types=[
            pltpu.VMEM((rows_per_w, 128), jnp.int32),
            pltpu.VMEM((epw, 128), jnp.float32),
            pltpu.VMEM((qpw, dim), jnp.float32),
            pltpu.VMEM((epw,), jnp.float32),
            pltpu.SemaphoreType.DMA,
        ])
    def gather_dot(src_hbm, dst_hbm, idx_hbm, out_hbm,
                   idx_v, rows_v, src_v, dots_v, sem):
        w = lax.axis_index("s") * _NC + lax.axis_index("c")
        pltpu.sync_copy(idx_hbm.at[w], idx_v)
        copies = [
            pltpu.async_copy(dst_hbm.at[idx_v.at[j]],
                             rows_v.at[pl.ds(j * 128, 128)], sem)
            for j in range(rows_per_w)
        ]
        pltpu.sync_copy(src_hbm.at[pl.ds(w * qpw, qpw)], src_v)
        for c in copies:
            c.wait()
        iota16 = lax.iota(jnp.int32, 16)

        def group(g, carry):
            e = g * 16 + iota16
            qloc = e // _K
            acc = jnp.zeros((16,), jnp.float32)
            for f in range(dim):
                fv = jnp.full((16,), f, jnp.int32)
                a = plsc.load_gather(rows_v, [e, fv])
                b = plsc.load_gather(src_v, [qloc, fv])
                acc = acc + a * b
            dots_v[pl.ds(g * 16, 16)] = acc
            return carry

        lax.fori_loop(0, epw // 16, group, 0)
        pltpu.sync_copy(dots_v, out_hbm.at[pl.ds(w * epw, epw)])

    @functools.partial(
        pl.kernel, mesh=mesh,
        compiler_params=pltpu.CompilerParams(needs_layout_passes=False),
        out_type=jax.ShapeDtypeStruct((n_edges,), jnp.float32),
        scratch_types=[
            pltpu.VMEM((n_edges,), jnp.float32),
            pltpu.VMEM((n_edges,), jnp.float32),
            pltpu.VMEM((16,), jnp.float32),
            pltpu.VMEM((16,), jnp.float32),
        ])
    def bn_exp_norm(dots_hbm, w_hbm, b_hbm, out_hbm, x_v, y_v, w_v, b_v):
        wid = lax.axis_index("s") * _NC + lax.axis_index("c")
        groups = n_edges // 16
        inv_n = jnp.float32(1.0 / n_edges)

        @pl.when(wid == 0)
        def _():
            pltpu.sync_copy(dots_hbm, x_v)
            pltpu.sync_copy(w_hbm, w_v)
            pltpu.sync_copy(b_hbm, b_v)

            def p_sum(i, s):
                return s + x_v[pl.ds(i * 16, 16)]

            total = jnp.sum(lax.fori_loop(
                0, groups, p_sum, jnp.zeros((16,), jnp.float32)))
            mean = jnp.full((16,), total, jnp.float32) * inv_n

            def p_var(i, s):
                v = x_v[pl.ds(i * 16, 16)] - mean
                return s + v * v

            vtot = jnp.sum(lax.fori_loop(
                0, groups, p_var, jnp.zeros((16,), jnp.float32)))
            var = jnp.full((16,), vtot, jnp.float32) * inv_n
            scale = _rsqrt16(var + _EPS) * w_v[...]
            bias = b_v[...]

            def p_exp(i, s):
                yv = jnp.exp((x_v[pl.ds(i * 16, 16)] - mean) * scale + bias)
                y_v[pl.ds(i * 16, 16)] = yv
                return s + yv

            es = lax.fori_loop(0, groups, p_exp, jnp.zeros((16,), jnp.float32))
            emean = jnp.full((16,), jnp.sum(es), jnp.float32) * inv_n
            winv = _recip16(emean)

            def p_norm(i, carry):
                y_v[pl.ds(i * 16, 16)] = y_v[pl.ds(i * 16, 16)] * winv
                return carry

            lax.fori_loop(0, groups, p_norm, 0)
            pltpu.sync_copy(y_v, out_hbm)

    return gather_dot, bn_exp_norm


def kernel(src_x, dst_x, src_batch, dst_batch, bn_weight, bn_bias):
    q, dim = src_x.shape
    n = dst_x.shape[0]
    nn_idx = _tc_topk(src_x, dst_x)
    dst_idx = nn_idx.reshape(-1)
    src_idx = jnp.repeat(jnp.arange(q, dtype=jnp.int32), _K)
    edge_index = jnp.stack([src_idx, dst_idx], axis=0)

    gather_dot, bn_exp_norm = _sc_kernels(q * _K, q, n, dim)
    idx2d = dst_idx.reshape(_NW, -1, 128)
    dst_pad = jnp.pad(dst_x, ((0, 0), (0, 128 - dim)))
    dots = gather_dot(src_x, dst_pad, idx2d)
    w16 = jnp.tile(bn_weight, 16)
    b16 = jnp.tile(bn_bias, 16)
    ew = bn_exp_norm(dots, w16, b16)
    return edge_index, ew


# speedup vs baseline: 7.4245x; 1.3462x over previous
import functools

import jax
import jax.numpy as jnp
import numpy as np
from jax import lax
from jax.experimental import pallas as pl
from jax.experimental.pallas import tpu as pltpu
from jax.experimental.pallas import tpu_sc as plsc

_K = 5
_EPS = 1e-5
_BQ = 256
_NC, _NS = 2, 16
_NW = _NC * _NS


_W = 128


def _topk_body(src_ref, dst_ref, idx_ref):
    bq = src_ref.shape[0]
    n = dst_ref.shape[0]
    nc = n // _W
    s = src_ref[...]
    d = dst_ref[...]
    mm = lax.dot_general(s, d, (((1,), (1,)), ((), ())),
                         preferred_element_type=jnp.float32)
    s2 = jnp.sum(s * s, axis=1, keepdims=True)
    d2 = jnp.sum(d * d, axis=1)
    cur = s2 - 2.0 * mm + d2[None, :]

    ciota = lax.broadcasted_iota(jnp.int32, (bq, nc), 1)
    pos_inf = jnp.float32(jnp.inf)
    m_mat = jnp.full((bq, nc), pos_inf)
    for c in range(nc):
        cmin = jnp.min(cur[:, c * _W:(c + 1) * _W], axis=1, keepdims=True)
        m_mat = jnp.where(ciota == c, cmin, m_mat)

    c5 = []
    for k in range(_K):
        mv = jnp.min(m_mat, axis=1, keepdims=True)
        ck = jnp.min(jnp.where(m_mat == mv, ciota, jnp.int32(nc)),
                     axis=1, keepdims=True)
        c5.append(ck)
        if k < _K - 1:
            m_mat = jnp.where(ciota == ck, pos_inf, m_mat)

    cvs = [jnp.zeros((bq, _W), jnp.float32) for _ in range(_K)]
    for c in range(nc):
        sl = cur[:, c * _W:(c + 1) * _W]
        for j in range(_K):
            cvs[j] = jnp.where(c5[j] == c, sl, cvs[j])
    cpt = jnp.concatenate(cvs, axis=1)

    fiota = lax.broadcasted_iota(jnp.int32, (bq, _K * _W), 1)
    cols = []
    for k in range(_K):
        mv = jnp.min(cpt, axis=1, keepdims=True)
        pos = jnp.min(jnp.where(cpt == mv, fiota, jnp.int32(_K * _W)),
                      axis=1, keepdims=True)
        j = lax.shift_right_logical(pos, 7)
        wloc = jnp.bitwise_and(pos, jnp.int32(_W - 1))
        cc = jnp.zeros_like(pos)
        for jj in range(_K):
            cc = jnp.where(j == jj, c5[jj], cc)
        cols.append(cc * _W + wloc)
        if k < _K - 1:
            cpt = jnp.where(fiota == pos, pos_inf, cpt)
    idx_ref[...] = jnp.concatenate(cols, axis=1)


def _tc_topk(src_x, dst_x):
    q, dim = src_x.shape
    n = dst_x.shape[0]
    return pl.pallas_call(
        _topk_body,
        grid=(q // _BQ,),
        in_specs=[
            pl.BlockSpec((_BQ, dim), lambda i: (i, 0)),
            pl.BlockSpec((n, dim), lambda i: (0, 0)),
        ],
        out_specs=pl.BlockSpec((_BQ, _K), lambda i: (i, 0)),
        out_shape=jax.ShapeDtypeStruct((q, _K), jnp.int32),
    )(src_x, dst_x)


def _rsqrt16(x):
    i = plsc.bitcast(x, jnp.int32)
    i = jnp.int32(0x5F3759DF) - lax.shift_right_logical(i, 1)
    y = plsc.bitcast(i, jnp.float32)
    for _ in range(4):
        y = y * (1.5 - 0.5 * x * y * y)
    return y


def _recip16(x):
    i = jnp.int32(0x7EF311C3) - plsc.bitcast(x, jnp.int32)
    y = plsc.bitcast(i, jnp.float32)
    for _ in range(4):
        y = y * (2.0 - x * y)
    return y


@functools.lru_cache(maxsize=None)
def _sc_kernels(n_edges, n_src, n_dst, dim):
    epw = n_edges // _NW
    qpw = n_src // _NW
    rows_per_w = epw // 128
    groups = epw // 16
    mesh = plsc.VectorSubcoreMesh(core_axis_name="c", subcore_axis_name="s",
                                  num_cores=_NC)
    inv_n = jnp.float32(1.0 / n_edges)

    @functools.partial(
        pl.kernel, mesh=mesh,
        compiler_params=pltpu.CompilerParams(needs_layout_passes=False),
        out_type=[jax.ShapeDtypeStruct((n_edges,), jnp.float32),
                  jax.ShapeDtypeStruct((2 * 16 * _NW,), jnp.float32)],
        scratch_types=[
            pltpu.VMEM((rows_per_w, 128), jnp.int32),
            pltpu.VMEM((epw, 128), jnp.float32),
            pltpu.VMEM((dim, 128), jnp.float32),
            pltpu.VMEM((epw,), jnp.float32),
            pltpu.VMEM((32,), jnp.float32),
            pltpu.SemaphoreType.DMA,
        ])
    def gather_dot(srct_hbm, dst_hbm, idx_hbm, out_hbm, part_hbm,
                   idx_v, rows_v, srct_v, dots_v, ps_v, sem):
        w = lax.axis_index("s") * _NC + lax.axis_index("c")
        pltpu.sync_copy(idx_hbm.at[w], idx_v)
        copies = [
            pltpu.async_copy(dst_hbm.at[idx_v.at[j]],
                             rows_v.at[pl.ds(j * 128, 128)], sem)
            for j in range(rows_per_w)
        ]
        pltpu.sync_copy(srct_hbm.at[:, pl.ds(w * qpw, qpw)], srct_v)
        for c in copies:
            c.wait()
        iota16 = lax.iota(jnp.int32, 16)
        iota5 = iota16 * _K
        fvs = [jnp.full((16,), f, jnp.int32) for f in range(dim)]

        def tgroup(t, carry):
            ssum, ssq = carry
            for j in range(rows_per_w):
                e = iota5 + (_K * 16 * t + j)
                acc = jnp.zeros((16,), jnp.float32)
                for f in range(dim):
                    a = plsc.load_gather(rows_v, [e, fvs[f]])
                    b = srct_v[f, pl.ds(16 * t, 16)]
                    acc = acc + a * b
                plsc.store_scatter(dots_v, [e], acc)
                ssum = ssum + acc
                ssq = ssq + acc * acc
            return ssum, ssq

        zero16 = jnp.zeros((16,), jnp.float32)
        ssum, ssq = lax.fori_loop(0, qpw // 16, tgroup, (zero16, zero16))
        ps_v[pl.ds(0, 16)] = ssum
        ps_v[pl.ds(16, 16)] = ssq
        pltpu.sync_copy(dots_v, out_hbm.at[pl.ds(w * epw, epw)])
        pltpu.sync_copy(ps_v.at[pl.ds(0, 16)],
                        part_hbm.at[pl.ds(w * 16, 16)])
        pltpu.sync_copy(ps_v.at[pl.ds(16, 16)],
                        part_hbm.at[pl.ds(16 * _NW + w * 16, 16)])

    epb = n_edges // _NS
    bgroups = epb // 16

    @functools.partial(
        pl.kernel, mesh=mesh,
        compiler_params=pltpu.CompilerParams(needs_layout_passes=False),
        out_type=jax.ShapeDtypeStruct((n_edges,), jnp.float32),
        scratch_types=[
            pltpu.VMEM((2 * 16 * _NW,), jnp.float32),
            pltpu.VMEM((epb,), jnp.float32),
            pltpu.VMEM((epb,), jnp.float32),
            pltpu.VMEM((16,), jnp.float32),
            pltpu.VMEM((16,), jnp.float32),
            pltpu.VMEM((16,), jnp.float32),
            pltpu.VMEM((_NS, 16), jnp.float32),
            pltpu.VMEM_SHARED((_NS, 16), jnp.float32),
        ])
    def bn_exp_norm(dots_hbm, part_in_hbm, w_hbm, b_hbm, out_hbm,
                    part_v, x_v, z_v, w_v, b_v, es_v, all_v, shr):
        core = lax.axis_index("c")
        sid = lax.axis_index("s")

        @pl.when(core == 0)
        def _():
            pltpu.sync_copy(part_in_hbm, part_v)
            pltpu.sync_copy(dots_hbm.at[pl.ds(sid * epb, epb)], x_v)
            pltpu.sync_copy(w_hbm, w_v)
            pltpu.sync_copy(b_hbm, b_v)
            sacc = jnp.zeros((16,), jnp.float32)
            qacc = jnp.zeros((16,), jnp.float32)
            for i in range(_NW):
                sacc = sacc + part_v[pl.ds(i * 16, 16)]
                qacc = qacc + part_v[pl.ds(16 * _NW + i * 16, 16)]
            mean = jnp.full((16,), jnp.sum(sacc), jnp.float32) * inv_n
            ex2 = jnp.full((16,), jnp.sum(qacc), jnp.float32) * inv_n
            var = ex2 - mean * mean
            scale = _rsqrt16(var + _EPS) * w_v[...]
            bias = b_v[...]

            def p_exp(g, es):
                zv = jnp.exp((x_v[pl.ds(g * 16, 16)] - mean) * scale + bias)
                z_v[pl.ds(g * 16, 16)] = zv
                return es + zv

            es = lax.fori_loop(0, bgroups, p_exp,
                               jnp.zeros((16,), jnp.float32))
            es_v[...] = es
            pltpu.sync_copy(es_v, shr.at[sid])
            plsc.subcore_barrier()
            pltpu.sync_copy(shr, all_v)
            tot = jnp.zeros((16,), jnp.float32)
            for i in range(_NS):
                tot = tot + all_v[i, :]
            emean = jnp.full((16,), jnp.sum(tot), jnp.float32) * inv_n
            winv = _recip16(emean)

            def p_norm(g, carry):
                z_v[pl.ds(g * 16, 16)] = z_v[pl.ds(g * 16, 16)] * winv
                return carry

            lax.fori_loop(0, bgroups, p_norm, 0)
            pltpu.sync_copy(z_v, out_hbm.at[pl.ds(sid * epb, epb)])

    return gather_dot, bn_exp_norm


def kernel(src_x, dst_x, src_batch, dst_batch, bn_weight, bn_bias):
    q, dim = src_x.shape
    n = dst_x.shape[0]
    nn_idx = _tc_topk(src_x, dst_x)
    dst_idx = nn_idx.reshape(-1)
    src_idx = jnp.asarray(np.repeat(np.arange(q, dtype=np.int32), _K))
    edge_index = jnp.stack([src_idx, dst_idx], axis=0)

    gather_dot, bn_exp_norm = _sc_kernels(q * _K, q, n, dim)
    idx2d = dst_idx.reshape(_NW, -1, 128)
    dst_pad = jnp.pad(dst_x, ((0, 0), (0, 128 - dim)))
    srct = src_x.T
    dots, part = gather_dot(srct, dst_pad, idx2d)
    w16 = jnp.tile(bn_weight, 16)
    b16 = jnp.tile(bn_bias, 16)
    ew = bn_exp_norm(dots, part, w16, b16)
    return edge_index, ew
